# baseline (speedup 1.0000x reference)
import jax
import jax.numpy as jnp
from jax.experimental import pallas as pl

_B, _C, _Q, _F = 8, 192, 256, 1024
_K = 192
_M = _Q - _K + 1
_FB = 128
_BB = 2


def _one_batch(qn, x):
    xn = x / jnp.maximum(jnp.sqrt(jnp.sum(x * x, axis=0, keepdims=True)), 1e-12)
    om = jnp.dot(qn, xn, preferred_element_type=jnp.float32)

    m = jnp.int32(_M)
    neg_base = jnp.int32(jnp.iinfo(jnp.int32).min) + jnp.int32(1 << 30)

    a_chunks = []
    for j in range(_F // _FB):
        omc = jax.lax.slice(om, (0, j * _FB), (_Q, (j + 1) * _FB))

        def _unkey(c):
            u = c ^ (jax.lax.shift_right_arithmetic(c, 31) & jnp.int32(0x7FFFFFFF))
            return jax.lax.bitcast_convert_type(u, jnp.float32)

        def _count_ge(cf):
            ind = jnp.where(omc >= cf, jnp.int32(1), jnp.int32(0))
            return jnp.sum(ind, axis=0, keepdims=True)

        cnt = _count_ge(jnp.zeros((1, _FB), jnp.float32))
        a = jnp.where(cnt >= m, jnp.int32(0), neg_base)
        a = jnp.broadcast_to(a, (1, _FB))
        for bit in range(29, -1, -1):
            c = a + jnp.int32(1 << bit)
            a = jnp.where(_count_ge(_unkey(c)) >= m, c, a)
        a_chunks.append(a)

    a = jnp.concatenate(a_chunks, axis=1)
    kth = jax.lax.bitcast_convert_type(
        a ^ (jax.lax.shift_right_arithmetic(a, 31) & jnp.int32(0x7FFFFFFF)),
        jnp.float32)

    keep = (om - kth) >= 0
    e = jnp.where(keep, jnp.exp(om - 1.0), 0.0)
    s = jnp.sum(e, axis=1, keepdims=True)
    acc = jax.lax.dot_general(
        e, xn, (((1,), (1,)), ((), ())), preferred_element_type=jnp.float32)
    return acc * (1.0 / s)


def _attn_kernel(qw_ref, x_ref, out_ref):
    qw = qw_ref[...]
    qn = qw / jnp.maximum(jnp.sqrt(jnp.sum(qw * qw, axis=1, keepdims=True)), 1e-12)
    for bb in range(_BB):
        out_ref[bb] = _one_batch(qn, x_ref[bb])


@jax.jit
def kernel(input_features, query_weight):
    x = input_features.reshape(_B, _C, _F)
    fn = pl.pallas_call(
        _attn_kernel,
        grid=(_B // _BB,),
        in_specs=[
            pl.BlockSpec((_Q, _C), lambda b: (0, 0)),
            pl.BlockSpec((_BB, _C, _F), lambda b: (b, 0, 0)),
        ],
        out_specs=pl.BlockSpec((_BB, _Q, _C), lambda b: (b, 0, 0)),
        out_shape=jax.ShapeDtypeStruct((_B, _Q, _C), jnp.float32),
    )
    return fn(query_weight, x)


# speedup vs baseline: 1.0673x; 1.0673x over previous
import jax
import jax.numpy as jnp
from jax.experimental import pallas as pl

_B, _C, _Q, _F = 8, 192, 256, 1024
_K = 192
_M = _Q - _K + 1
_FB = 128
_BB = 2


def _one_batch(qn, x):
    xn = x / jnp.maximum(jnp.sqrt(jnp.sum(x * x, axis=0, keepdims=True)), 1e-12)
    om = jnp.dot(qn, xn, preferred_element_type=jnp.float32)

    m = jnp.int32(_M)
    neg_base = jnp.int32(jnp.iinfo(jnp.int32).min) + jnp.int32(1 << 30)

    a_chunks = []
    for j in range(_F // _FB):
        omc = jax.lax.slice(om, (0, j * _FB), (_Q, (j + 1) * _FB))
        i = jax.lax.bitcast_convert_type(omc, jnp.int32)
        key = i ^ (jax.lax.shift_right_arithmetic(i, 31) & jnp.int32(0x7FFFFFFF))

        def _count_ge(c):
            ind = jnp.where(key >= c, jnp.int32(1), jnp.int32(0))
            return jnp.sum(ind, axis=0, keepdims=True)

        cnt = _count_ge(jnp.zeros((1, _FB), jnp.int32))
        a = jnp.where(cnt >= m, jnp.int32(0), neg_base)
        a = jnp.broadcast_to(a, (1, _FB))
        for bit in range(29, -1, -1):
            c = a + jnp.int32(1 << bit)
            a = jnp.where(_count_ge(c) >= m, c, a)
        a_chunks.append(a)

    a = jnp.concatenate(a_chunks, axis=1)
    kth = jax.lax.bitcast_convert_type(
        a ^ (jax.lax.shift_right_arithmetic(a, 31) & jnp.int32(0x7FFFFFFF)),
        jnp.float32)

    keep = (om - kth) >= 0
    e = jnp.where(keep, jnp.exp(om - 1.0), 0.0)
    s = jnp.sum(e, axis=1, keepdims=True)
    acc = jax.lax.dot_general(
        e, xn, (((1,), (1,)), ((), ())), preferred_element_type=jnp.float32)
    return acc * (1.0 / s)


def _attn_kernel(qw_ref, x_ref, out_ref):
    qw = qw_ref[...]
    qn = qw / jnp.maximum(jnp.sqrt(jnp.sum(qw * qw, axis=1, keepdims=True)), 1e-12)
    for bb in range(_BB):
        out_ref[bb] = _one_batch(qn, x_ref[bb])


@jax.jit
def kernel(input_features, query_weight):
    x = input_features.reshape(_B, _C, _F)
    fn = pl.pallas_call(
        _attn_kernel,
        grid=(_B // _BB,),
        in_specs=[
            pl.BlockSpec((_Q, _C), lambda b: (0, 0)),
            pl.BlockSpec((_BB, _C, _F), lambda b: (b, 0, 0)),
        ],
        out_specs=pl.BlockSpec((_BB, _Q, _C), lambda b: (b, 0, 0)),
        out_shape=jax.ShapeDtypeStruct((_B, _Q, _C), jnp.float32),
    )
    return fn(query_weight, x)


# speedup vs baseline: 1.0760x; 1.0081x over previous
import jax
import jax.numpy as jnp
from jax.experimental import pallas as pl

_B, _C, _Q, _F = 8, 192, 256, 1024
_K = 192
_M = _Q - _K + 1
_FB = 128
_BB = 4


def _one_batch(qn, x):
    xn = x / jnp.maximum(jnp.sqrt(jnp.sum(x * x, axis=0, keepdims=True)), 1e-12)
    om = jnp.dot(qn, xn, preferred_element_type=jnp.float32)

    m = jnp.int32(_M)
    neg_base = jnp.int32(jnp.iinfo(jnp.int32).min) + jnp.int32(1 << 30)

    a_chunks = []
    for j in range(_F // _FB):
        omc = jax.lax.slice(om, (0, j * _FB), (_Q, (j + 1) * _FB))
        i = jax.lax.bitcast_convert_type(omc, jnp.int32)
        key = i ^ (jax.lax.shift_right_arithmetic(i, 31) & jnp.int32(0x7FFFFFFF))

        def _count_ge(c):
            ind = jnp.where(key >= c, jnp.int32(1), jnp.int32(0))
            return jnp.sum(ind, axis=0, keepdims=True)

        cnt = _count_ge(jnp.zeros((1, _FB), jnp.int32))
        a = jnp.where(cnt >= m, jnp.int32(0), neg_base)
        a = jnp.broadcast_to(a, (1, _FB))
        for bit in range(29, -1, -1):
            c = a + jnp.int32(1 << bit)
            a = jnp.where(_count_ge(c) >= m, c, a)
        a_chunks.append(a)

    a = jnp.concatenate(a_chunks, axis=1)
    kth = jax.lax.bitcast_convert_type(
        a ^ (jax.lax.shift_right_arithmetic(a, 31) & jnp.int32(0x7FFFFFFF)),
        jnp.float32)

    keep = (om - kth) >= 0
    e = jnp.where(keep, jnp.exp(om - 1.0), 0.0)
    s = jnp.sum(e, axis=1, keepdims=True)
    acc = jax.lax.dot_general(
        e, xn, (((1,), (1,)), ((), ())), preferred_element_type=jnp.float32)
    return acc * (1.0 / s)


def _attn_kernel(qw_ref, x_ref, out_ref):
    qw = qw_ref[...]
    qn = qw / jnp.maximum(jnp.sqrt(jnp.sum(qw * qw, axis=1, keepdims=True)), 1e-12)
    for bb in range(_BB):
        out_ref[bb] = _one_batch(qn, x_ref[bb])


@jax.jit
def kernel(input_features, query_weight):
    x = input_features.reshape(_B, _C, _F)
    fn = pl.pallas_call(
        _attn_kernel,
        grid=(_B // _BB,),
        in_specs=[
            pl.BlockSpec((_Q, _C), lambda b: (0, 0)),
            pl.BlockSpec((_BB, _C, _F), lambda b: (b, 0, 0)),
        ],
        out_specs=pl.BlockSpec((_BB, _Q, _C), lambda b: (b, 0, 0)),
        out_shape=jax.ShapeDtypeStruct((_B, _Q, _C), jnp.float32),
    )
    return fn(query_weight, x)
